# baseline (speedup 1.0000x reference)
import functools

import jax
import jax.numpy as jnp
from jax import lax
from jax.experimental import pallas as pl
from jax.experimental.pallas import tpu as pltpu
from jax.experimental.pallas import tpu_sc as plsc

NUM_PATCHES = 576
PROJECTION_DIM = 768
BATCH = 64

TC_BATCH = 44
SC_BATCH = BATCH - TC_BATCH
TC_BB = 4

NC = 2
NS = 16
NW = NC * NS
LANES = 16
COLV = PROJECTION_DIM // LANES

NBG = 4
NTG = NW // NBG
BPW = SC_BATCH // NBG
PPW = NUM_PATCHES // NTG
CP = 24
SPB = PPW // CP
NSTEP = BPW * SPB


def _tc_add(x_ref, pos_ref, o_ref):
    o_ref[...] = x_ref[...] + pos_ref[...]


def _sc_add(x_hbm, pos_hbm, out_hbm, p_v, x_v, o_v, in_sem, out_sem):
    wid = lax.axis_index("s") * NC + lax.axis_index("c")
    bg = lax.div(wid, NTG)
    tg = lax.rem(wid, NTG)
    b0 = bg * BPW
    p0 = tg * PPW

    pltpu.sync_copy(pos_hbm.at[pl.ds(p0, PPW)], p_v)

    def x_slice(s):
        b = TC_BATCH + b0 + lax.div(s, SPB)
        poff = p0 + lax.rem(s, SPB) * CP
        return x_hbm.at[b, pl.ds(poff, CP)]

    def out_slice(s):
        b = b0 + lax.div(s, SPB)
        poff = p0 + lax.rem(s, SPB) * CP
        return out_hbm.at[b, pl.ds(poff, CP)]

    pltpu.async_copy(x_slice(0), x_v.at[0], in_sem.at[0])

    def step_body(s, _):
        slot = lax.rem(s, 2)
        nslot = lax.rem(s + 1, 2)

        @pl.when(s + 1 < NSTEP)
        def _start_next_in():
            pltpu.async_copy(x_slice(s + 1), x_v.at[nslot], in_sem.at[nslot])

        pltpu.make_async_copy(x_slice(s), x_v.at[slot], in_sem.at[slot]).wait()

        @pl.when(s >= 2)
        def _drain_prev_out():
            pltpu.make_async_copy(o_v.at[slot], out_slice(s - 2),
                                  out_sem.at[slot]).wait()

        prow = lax.rem(s, SPB) * CP

        @plsc.parallel_loop(0, CP)
        def _row(r):
            for k in range(COLV):
                sl = pl.ds(k * LANES, LANES)
                o_v[slot, r, sl] = x_v[slot, r, sl] + p_v[prow + r, sl]

        pltpu.async_copy(o_v.at[slot], out_slice(s), out_sem.at[slot])
        return ()

    lax.fori_loop(0, NSTEP, step_body, ())

    for s in (NSTEP - 2, NSTEP - 1):
        slot = s % 2
        pltpu.make_async_copy(o_v.at[slot], out_slice(s),
                              out_sem.at[slot]).wait()


def kernel(x, pos_table):
    mesh = plsc.VectorSubcoreMesh(core_axis_name="c", subcore_axis_name="s")
    sc_run = functools.partial(
        pl.kernel,
        out_type=jax.ShapeDtypeStruct((SC_BATCH,) + x.shape[1:], jnp.float32),
        mesh=mesh,
        scratch_types=[
            pltpu.VMEM((PPW, PROJECTION_DIM), jnp.float32),
            pltpu.VMEM((2, CP, PROJECTION_DIM), jnp.float32),
            pltpu.VMEM((2, CP, PROJECTION_DIM), jnp.float32),
            pltpu.SemaphoreType.DMA((2,)),
            pltpu.SemaphoreType.DMA((2,)),
        ],
    )(_sc_add)
    sc_out = sc_run(x, pos_table)

    tc_out = pl.pallas_call(
        _tc_add,
        grid=(TC_BATCH // TC_BB,),
        in_specs=[
            pl.BlockSpec((TC_BB, NUM_PATCHES, PROJECTION_DIM),
                         lambda b: (b, 0, 0)),
            pl.BlockSpec((NUM_PATCHES, PROJECTION_DIM), lambda b: (0, 0)),
        ],
        out_specs=pl.BlockSpec((TC_BB, NUM_PATCHES, PROJECTION_DIM),
                               lambda b: (b, 0, 0)),
        out_shape=jax.ShapeDtypeStruct((TC_BATCH,) + x.shape[1:], x.dtype),
    )(x[:TC_BATCH], pos_table)

    return jnp.concatenate([tc_out, sc_out], axis=0)


# speedup vs baseline: 1.3661x; 1.3661x over previous
import functools

import jax
import jax.numpy as jnp
from jax import lax
from jax.experimental import pallas as pl
from jax.experimental.pallas import tpu as pltpu
from jax.experimental.pallas import tpu_sc as plsc

NUM_PATCHES = 576
PROJECTION_DIM = 768
BATCH = 64

TC_BATCH = 0
SC_BATCH = BATCH - TC_BATCH

NC = 2
NS = 16
NW = NC * NS
LANES = 16
COLV = PROJECTION_DIM // LANES

NBG = 4
NTG = NW // NBG
BPW = SC_BATCH // NBG
PPW = NUM_PATCHES // NTG
CP = 8
SPB = PPW // CP
NSTEP = BPW * SPB


def _tc_add(x_ref, pos_ref, o_ref):
    o_ref[...] = x_ref[...] + pos_ref[...]


def _sc_add(x_hbm, pos_hbm, out_hbm, p_v, x_v, o_v, in_sem, out_sem):
    wid = lax.axis_index("s") * NC + lax.axis_index("c")
    bg = lax.div(wid, NTG)
    tg = lax.rem(wid, NTG)
    b0 = bg * BPW
    p0 = tg * PPW

    pltpu.sync_copy(pos_hbm.at[pl.ds(p0, PPW)], p_v)

    def x_slice(s):
        b = TC_BATCH + b0 + lax.div(s, SPB)
        poff = p0 + lax.rem(s, SPB) * CP
        return x_hbm.at[b, pl.ds(poff, CP)]

    def out_slice(s):
        b = b0 + lax.div(s, SPB)
        poff = p0 + lax.rem(s, SPB) * CP
        return out_hbm.at[b, pl.ds(poff, CP)]

    pltpu.async_copy(x_slice(0), x_v.at[0], in_sem.at[0])

    def step_body(s, _):
        slot = lax.rem(s, 2)
        nslot = lax.rem(s + 1, 2)

        @pl.when(s + 1 < NSTEP)
        def _start_next_in():
            pltpu.async_copy(x_slice(s + 1), x_v.at[nslot], in_sem.at[nslot])

        pltpu.make_async_copy(x_slice(s), x_v.at[slot], in_sem.at[slot]).wait()

        @pl.when(s >= 2)
        def _drain_prev_out():
            pltpu.make_async_copy(o_v.at[slot], out_slice(s - 2),
                                  out_sem.at[slot]).wait()

        prow = lax.rem(s, SPB) * CP

        @plsc.parallel_loop(0, CP)
        def _row(r):
            for k in range(COLV):
                sl = pl.ds(k * LANES, LANES)
                o_v[slot, r, sl] = x_v[slot, r, sl] + p_v[prow + r, sl]

        pltpu.async_copy(o_v.at[slot], out_slice(s), out_sem.at[slot])
        return ()

    lax.fori_loop(0, NSTEP, step_body, ())

    for s in (NSTEP - 2, NSTEP - 1):
        slot = s % 2
        pltpu.make_async_copy(o_v.at[slot], out_slice(s),
                              out_sem.at[slot]).wait()


def kernel(x, pos_table):
    mesh = plsc.VectorSubcoreMesh(core_axis_name="c", subcore_axis_name="s")
    sc_run = functools.partial(
        pl.kernel,
        out_type=jax.ShapeDtypeStruct((SC_BATCH,) + x.shape[1:], jnp.float32),
        mesh=mesh,
        scratch_types=[
            pltpu.VMEM((PPW, PROJECTION_DIM), jnp.float32),
            pltpu.VMEM((2, CP, PROJECTION_DIM), jnp.float32),
            pltpu.VMEM((2, CP, PROJECTION_DIM), jnp.float32),
            pltpu.SemaphoreType.DMA((2,)),
            pltpu.SemaphoreType.DMA((2,)),
        ],
    )(_sc_add)
    return sc_run(x, pos_table)


# speedup vs baseline: 1.8898x; 1.3833x over previous
import functools

import jax
import jax.numpy as jnp
from jax import lax
from jax.experimental import pallas as pl
from jax.experimental.pallas import tpu as pltpu
from jax.experimental.pallas import tpu_sc as plsc

NUM_PATCHES = 576
PROJECTION_DIM = 768
BATCH = 64

TC_BATCH = 0
SC_BATCH = BATCH - TC_BATCH

NC = 2
NS = 16
NW = NC * NS
LANES = 16
COLV = PROJECTION_DIM // LANES

NBG = 4
NTG = NW // NBG
BPW = SC_BATCH // NBG
PPW = NUM_PATCHES // NTG
CP = 24
SPB = PPW // CP
NSTEP = BPW * SPB


def _tc_add(x_ref, pos_ref, o_ref):
    o_ref[...] = x_ref[...] + pos_ref[...]


def _sc_add(x_hbm, pos_hbm, out_hbm, p_v, x_v, o_v, in_sem, out_sem):
    wid = lax.axis_index("s") * NC + lax.axis_index("c")
    bg = lax.div(wid, NTG)
    tg = lax.rem(wid, NTG)
    b0 = bg * BPW
    p0 = tg * PPW

    pltpu.sync_copy(pos_hbm.at[pl.ds(p0, PPW)], p_v)

    def x_slice(s):
        b = TC_BATCH + b0 + lax.div(s, SPB)
        poff = p0 + lax.rem(s, SPB) * CP
        return x_hbm.at[b, pl.ds(poff, CP)]

    def out_slice(s):
        b = b0 + lax.div(s, SPB)
        poff = p0 + lax.rem(s, SPB) * CP
        return out_hbm.at[b, pl.ds(poff, CP)]

    pltpu.async_copy(x_slice(0), x_v.at[0], in_sem.at[0])

    def step_body(s, _):
        slot = lax.rem(s, 2)
        nslot = lax.rem(s + 1, 2)

        @pl.when(s + 1 < NSTEP)
        def _start_next_in():
            pltpu.async_copy(x_slice(s + 1), x_v.at[nslot], in_sem.at[nslot])

        pltpu.make_async_copy(x_slice(s), x_v.at[slot], in_sem.at[slot]).wait()

        @pl.when(s >= 2)
        def _drain_prev_out():
            pltpu.make_async_copy(o_v.at[slot], out_slice(s - 2),
                                  out_sem.at[slot]).wait()

        prow = lax.rem(s, SPB) * CP

        @plsc.parallel_loop(0, CP)
        def _row(r):
            for k in range(COLV):
                sl = pl.ds(k * LANES, LANES)
                o_v[slot, r, sl] = x_v[slot, r, sl] + p_v[prow + r, sl]

        pltpu.async_copy(o_v.at[slot], out_slice(s), out_sem.at[slot])
        return ()

    lax.fori_loop(0, NSTEP, step_body, ())

    for s in (NSTEP - 2, NSTEP - 1):
        slot = s % 2
        pltpu.make_async_copy(o_v.at[slot], out_slice(s),
                              out_sem.at[slot]).wait()


def kernel(x, pos_table):
    mesh = plsc.VectorSubcoreMesh(core_axis_name="c", subcore_axis_name="s")
    sc_run = functools.partial(
        pl.kernel,
        out_type=jax.ShapeDtypeStruct((SC_BATCH,) + x.shape[1:], jnp.float32),
        mesh=mesh,
        scratch_types=[
            pltpu.VMEM((PPW, PROJECTION_DIM), jnp.float32),
            pltpu.VMEM((2, CP, PROJECTION_DIM), jnp.float32),
            pltpu.VMEM((2, CP, PROJECTION_DIM), jnp.float32),
            pltpu.SemaphoreType.DMA((2,)),
            pltpu.SemaphoreType.DMA((2,)),
        ],
    )(_sc_add)
    return sc_run(x, pos_table)


# speedup vs baseline: 1.8904x; 1.0003x over previous
import functools

import jax
import jax.numpy as jnp
from jax import lax
from jax.experimental import pallas as pl
from jax.experimental.pallas import tpu as pltpu
from jax.experimental.pallas import tpu_sc as plsc

NUM_PATCHES = 576
PROJECTION_DIM = 768
BATCH = 64

NC = 2
NS = 16
NW = NC * NS
LANES = 16
COLV = PROJECTION_DIM // LANES

NBG = 4
NTG = NW // NBG
BPW = BATCH // NBG
PPW = NUM_PATCHES // NTG
CP = 24
SPB = PPW // CP
NSTEP = BPW * SPB


def _sc_add(x_hbm, pos_hbm, out_hbm, p_v, x_v, o_v, in_sem, out_sem):
    wid = lax.axis_index("s") * NC + lax.axis_index("c")
    bg = lax.div(wid, NTG)
    tg = lax.rem(wid, NTG)
    b0 = bg * BPW
    p0 = tg * PPW

    pltpu.sync_copy(pos_hbm.at[pl.ds(p0, PPW)], p_v)

    def x_slice(s):
        b = b0 + lax.div(s, SPB)
        poff = p0 + lax.rem(s, SPB) * CP
        return x_hbm.at[b, pl.ds(poff, CP)]

    def out_slice(s):
        b = b0 + lax.div(s, SPB)
        poff = p0 + lax.rem(s, SPB) * CP
        return out_hbm.at[b, pl.ds(poff, CP)]

    pltpu.async_copy(x_slice(0), x_v.at[0], in_sem.at[0])

    def step_body(s, _):
        slot = lax.rem(s, 2)
        nslot = lax.rem(s + 1, 2)

        @pl.when(s + 1 < NSTEP)
        def _start_next_in():
            pltpu.async_copy(x_slice(s + 1), x_v.at[nslot], in_sem.at[nslot])

        pltpu.make_async_copy(x_slice(s), x_v.at[slot], in_sem.at[slot]).wait()

        @pl.when(s >= 2)
        def _drain_prev_out():
            pltpu.make_async_copy(o_v.at[slot], out_slice(s - 2),
                                  out_sem.at[slot]).wait()

        prow = lax.rem(s, SPB) * CP

        @plsc.parallel_loop(0, CP)
        def _row(r):
            for k in range(COLV):
                sl = pl.ds(k * LANES, LANES)
                o_v[slot, r, sl] = x_v[slot, r, sl] + p_v[prow + r, sl]

        pltpu.async_copy(o_v.at[slot], out_slice(s), out_sem.at[slot])
        return ()

    lax.fori_loop(0, NSTEP, step_body, ())

    for s in (NSTEP - 2, NSTEP - 1):
        slot = s % 2
        pltpu.make_async_copy(o_v.at[slot], out_slice(s),
                              out_sem.at[slot]).wait()


def kernel(x, pos_table):
    mesh = plsc.VectorSubcoreMesh(core_axis_name="c", subcore_axis_name="s")
    sc_run = functools.partial(
        pl.kernel,
        out_type=jax.ShapeDtypeStruct(x.shape, jnp.float32),
        mesh=mesh,
        scratch_types=[
            pltpu.VMEM((PPW, PROJECTION_DIM), jnp.float32),
            pltpu.VMEM((2, CP, PROJECTION_DIM), jnp.float32),
            pltpu.VMEM((2, CP, PROJECTION_DIM), jnp.float32),
            pltpu.SemaphoreType.DMA((2,)),
            pltpu.SemaphoreType.DMA((2,)),
        ],
    )(_sc_add)
    return sc_run(x, pos_table)


# speedup vs baseline: 1.9044x; 1.0074x over previous
import functools

import jax
import jax.numpy as jnp
from jax import lax
from jax.experimental import pallas as pl
from jax.experimental.pallas import tpu as pltpu
from jax.experimental.pallas import tpu_sc as plsc

NUM_PATCHES = 576
PROJECTION_DIM = 768
BATCH = 64

NC = 2
NS = 16
NW = NC * NS
LANES = 16
COLV = PROJECTION_DIM // LANES

NBG = 4
NTG = NW // NBG
BPW = BATCH // NBG
PPW = NUM_PATCHES // NTG
CP = 24
SPB = PPW // CP
NSTEP = BPW * SPB


def _sc_add(x_hbm, pos_hbm, out_hbm, p_v, x_v, o_v, in_sem, out_sem):
    wid = lax.axis_index("s") * NC + lax.axis_index("c")
    bg = lax.div(wid, NTG)
    tg = lax.rem(wid, NTG)
    b0 = bg * BPW
    p0 = tg * PPW

    def x_slice(s):
        b = b0 + lax.div(s, SPB)
        poff = p0 + lax.rem(s, SPB) * CP
        return x_hbm.at[b, pl.ds(poff, CP)]

    def out_slice(s):
        b = b0 + lax.div(s, SPB)
        poff = p0 + lax.rem(s, SPB) * CP
        return out_hbm.at[b, pl.ds(poff, CP)]

    pltpu.async_copy(x_slice(0), x_v.at[0], in_sem.at[0])
    pltpu.sync_copy(pos_hbm.at[pl.ds(p0, PPW)], p_v)

    def step_body(s, _):
        slot = lax.rem(s, 2)
        nslot = lax.rem(s + 1, 2)

        @pl.when(s + 1 < NSTEP)
        def _start_next_in():
            pltpu.async_copy(x_slice(s + 1), x_v.at[nslot], in_sem.at[nslot])

        pltpu.make_async_copy(x_slice(s), x_v.at[slot], in_sem.at[slot]).wait()

        @pl.when(s >= 2)
        def _drain_prev_out():
            pltpu.make_async_copy(o_v.at[slot], out_slice(s - 2),
                                  out_sem.at[slot]).wait()

        prow = lax.rem(s, SPB) * CP

        @plsc.parallel_loop(0, CP)
        def _row(r):
            for k in range(COLV):
                sl = pl.ds(k * LANES, LANES)
                o_v[slot, r, sl] = x_v[slot, r, sl] + p_v[prow + r, sl]

        pltpu.async_copy(o_v.at[slot], out_slice(s), out_sem.at[slot])
        return ()

    lax.fori_loop(0, NSTEP, step_body, ())

    for s in (NSTEP - 2, NSTEP - 1):
        slot = s % 2
        pltpu.make_async_copy(o_v.at[slot], out_slice(s),
                              out_sem.at[slot]).wait()


def kernel(x, pos_table):
    mesh = plsc.VectorSubcoreMesh(core_axis_name="c", subcore_axis_name="s")
    sc_run = functools.partial(
        pl.kernel,
        out_type=jax.ShapeDtypeStruct(x.shape, jnp.float32),
        mesh=mesh,
        scratch_types=[
            pltpu.VMEM((PPW, PROJECTION_DIM), jnp.float32),
            pltpu.VMEM((2, CP, PROJECTION_DIM), jnp.float32),
            pltpu.VMEM((2, CP, PROJECTION_DIM), jnp.float32),
            pltpu.SemaphoreType.DMA((2,)),
            pltpu.SemaphoreType.DMA((2,)),
        ],
    )(_sc_add)
    return sc_run(x, pos_table)
